# Initial kernel scaffold; baseline (speedup 1.0000x reference)
import jax
import jax.numpy as jnp
from jax.experimental import pallas as pl


def kernel(src, bgr, pha, fgr, err, hid, conv1_w, bn1_g, bn1_b, bn1_m, bn1_v, conv2_w, bn2_g, bn2_b, bn2_m, bn2_v, conv3_w, bn3_g, bn3_b, bn3_m, bn3_v, conv4_w, conv4_b):
    raise NotImplementedError("write your pallas kernel here")


# speedup vs baseline: 289.8384x; 289.8384x over previous
import numpy as np
import jax
import jax.numpy as jnp
from jax.experimental import pallas as pl

BNEPS = 1e-05
KSEL = 5000
NT = 4
TILE = 4424
TIN = TILE + 136
FL = 17832


def _fold_bn(w, g, b, m, v):
    s = g / jnp.sqrt(v + BNEPS)
    return w * s[:, None, None, None], b - m * s


def _poly_sel(f, o):
    A = np.zeros((2, 2, f, f, f, f, 3, 3), np.float32)
    smin = o // f
    for py in range(f):
        for dy in range(3):
            vy = py + dy + o
            sy, qy = vy // f - smin, vy % f
            for px in range(f):
                for dx in range(3):
                    vx = px + dx + o
                    sx, qx = vx // f - smin, vx % f
                    A[sy, sx, qy, qx, py, px, dy, dx] = 1.0
    return A.reshape(4, f, f, f, f, 3, 3)


_A1 = _poly_sel(2, -2)
_A2 = _poly_sel(2, 0)
_A3 = _poly_sel(4, -2)
_A4 = _poly_sel(4, 0)
_F3 = np.zeros((160, 352), np.float32)
for _qy in range(4):
    for _qx in range(4):
        _r = (_qy * 4 + _qx) * 22
        for _c in range(16):
            _F3[16 * (2 * (_qy // 2) + (_qx // 2)) + _c, _r + _c] = 1.0
        for _c in range(6):
            _F3[64 + 6 * (4 * _qy + _qx) + _c, _r + 16 + _c] = 1.0
_F3 = jnp.asarray(_F3)


def _poly_w(wf, A, f):
    W = jnp.einsum('tabcdyx,oiyx->tabicdo', jnp.asarray(A), wf)
    return W.reshape(4, f * f * wf.shape[1], f * f * wf.shape[0])


def _flat(content, rpad, cpad):
    B, H, W, C = content.shape
    x = jnp.pad(content, ((0, 0), (rpad, 133 - rpad - H), (cpad, 132 - cpad - W), (0, 0)))
    x = x.reshape(B, 133 * 132, C)
    return jnp.pad(x, ((0, 0), (0, FL - 133 * 132), (0, 0)))


def _tiles(xf):
    return jnp.stack([xf[:, t * TILE:t * TILE + TIN] for t in range(NT)], axis=1)


def _topk_kernel(e_ref, o_ref):
    e = e_ref[0]
    bits = jax.lax.bitcast_convert_type(e, jnp.int32)

    def body(i, lohi):
        lo, hi = lohi
        mid = lo + (hi - lo) // 2
        cnt = jnp.sum((bits > mid).astype(jnp.int32))
        big = cnt >= KSEL
        return jnp.where(big, mid + 1, lo), jnp.where(big, hi, mid)

    t, _ = jax.lax.fori_loop(0, 31, body, (jnp.int32(0), jnp.int32(2**31 - 1)))
    gt = bits > t
    eq = bits == t
    need = (KSEL - jnp.sum(gt.astype(jnp.int32))).astype(jnp.float32)
    eqf = eq.astype(jnp.float32)
    row = jax.lax.broadcasted_iota(jnp.int32, (128, 128), 0)
    col = jax.lax.broadcasted_iota(jnp.int32, (128, 128), 1)
    lt = (row < col).astype(jnp.float32)
    sl = (row > col).astype(jnp.float32)
    in_row = jax.lax.dot_general(eqf, lt, (((1,), (0,)), ((), ())),
                                 preferred_element_type=jnp.float32)
    rowcnt = jnp.sum(eqf, axis=1, keepdims=True)
    rowpref = jax.lax.dot_general(sl, rowcnt, (((1,), (0,)), ((), ())),
                                  preferred_element_type=jnp.float32)
    prefix = in_row + rowpref
    mask = gt | (eq & (prefix < need))
    o_ref[0] = jnp.where(mask & (bits > 0), 1.0, 0.0)


def _prep_kernel(xp_ref, sb_ref, o_ref):
    Pw = xp_ref[0, 0]
    r0 = 0.25 * Pw[0:16] + 0.75 * Pw[1:17]
    r1 = 0.75 * Pw[1:17] + 0.25 * Pw[2:18]
    S = sb_ref[0]
    for qy in range(2):
        r = (r0, r1)[qy]
        c0 = 0.25 * r[:, 0:128, :] + 0.75 * r[:, 1:129, :]
        c1 = 0.75 * r[:, 1:129, :] + 0.25 * r[:, 2:130, :]
        for qx in range(2):
            u = (c0, c1)[qx]
            acc = None
            for a in range(2):
                for b in range(2):
                    cc = ((2 * qy + a) * 4 + (2 * qx + b)) * 6
                    piece = S[:, :, cc:cc + 6]
                    acc = piece if acc is None else acc + piece
            base = (qy * 2 + qx) * 42
            o_ref[0, :, :, base:base + 36] = u
            o_ref[0, :, :, base + 36:base + 42] = 0.25 * acc


def _mk_conv(relu, two_in):
    if two_in:
        def k(xa_ref, xb_ref, wa_ref, wb_ref, b_ref, o_ref):
            acc = None
            for j, off in enumerate((0, 1, 132, 133)):
                xa = xa_ref[0, 0, off:off + TILE, :]
                xb = xb_ref[0, 0, off:off + TILE, :]
                p = jax.lax.dot_general(xa, wa_ref[j], (((1,), (0,)), ((), ())),
                                        preferred_element_type=jnp.float32)
                p = p + jax.lax.dot_general(xb, wb_ref[j], (((1,), (0,)), ((), ())),
                                            preferred_element_type=jnp.float32)
                acc = p if acc is None else acc + p
            acc = acc + b_ref[0]
            o_ref[0] = jnp.maximum(acc, 0.0) if relu else acc
        return k

    def k(x_ref, w_ref, b_ref, o_ref):
        acc = None
        for j, off in enumerate((0, 1, 132, 133)):
            xs = x_ref[0, 0, off:off + TILE, :]
            p = jax.lax.dot_general(xs, w_ref[j], (((1,), (0,)), ((), ())),
                                    preferred_element_type=jnp.float32)
            acc = p if acc is None else acc + p
        acc = acc + b_ref[0]
        o_ref[0] = jnp.maximum(acc, 0.0) if relu else acc
    return k


def _merge_kernel(h4_ref, pf_ref, ref_ref, o_ref):
    h4 = h4_ref[0, :, 0:128, :]
    P = pf_ref[0, 0]
    b0 = 0.375 * P[0:16] + 0.625 * P[1:17]
    b1 = 0.125 * P[0:16] + 0.875 * P[1:17]
    b2 = 0.875 * P[1:17] + 0.125 * P[2:18]
    b3 = 0.625 * P[1:17] + 0.375 * P[2:18]
    sel = ref_ref[0] > 0.0
    for ty in range(4):
        r = (b0, b1, b2, b3)[ty]
        cs = (0.375 * r[:, 0:128, :] + 0.625 * r[:, 1:129, :],
              0.125 * r[:, 0:128, :] + 0.875 * r[:, 1:129, :],
              0.875 * r[:, 1:129, :] + 0.125 * r[:, 2:130, :],
              0.625 * r[:, 1:129, :] + 0.375 * r[:, 2:130, :])
        for tx in range(4):
            cc = (ty * 4 + tx) * 4
            o_ref[0, :, :, cc:cc + 4] = jnp.where(sel, h4[:, :, cc:cc + 4], cs[tx])


def _conv_call(xf, W, bias, cout, relu):
    B = xf.shape[0]
    xt = _tiles(xf)
    return pl.pallas_call(
        _mk_conv(relu, False),
        grid=(B, NT),
        in_specs=[
            pl.BlockSpec((1, 1, TIN, xt.shape[3]), lambda b, t: (b, t, 0, 0)),
            pl.BlockSpec(W.shape, lambda b, t: (0, 0, 0)),
            pl.BlockSpec((1, cout), lambda b, t: (0, 0)),
        ],
        out_specs=pl.BlockSpec((1, TILE, cout), lambda b, t: (b, t, 0)),
        out_shape=jax.ShapeDtypeStruct((B, NT * TILE, cout), jnp.float32),
    )(xt, W, bias)


def kernel(src, bgr, pha, fgr, err, hid, conv1_w, bn1_g, bn1_b, bn1_m, bn1_v,
           conv2_w, bn2_g, bn2_b, bn2_m, bn2_v, conv3_w, bn3_g, bn3_b, bn3_m, bn3_v,
           conv4_w, conv4_b):
    B = src.shape[0]
    f32 = jnp.float32
    nhwc = lambda a: jnp.transpose(a, (0, 2, 3, 1))

    x36 = jnp.concatenate([nhwc(hid), nhwc(pha), nhwc(fgr)], axis=-1)
    x36p = jnp.pad(x36, ((0, 0), (1, 1), (1, 1), (0, 0)), mode='edge')
    x36t = jnp.stack([x36p[:, 16 * t:16 * t + 18] for t in range(8)], axis=1)
    sb = jnp.concatenate([src, bgr], axis=1)
    sb16 = jnp.transpose(sb.reshape(B, 6, 128, 4, 128, 4),
                         (0, 2, 4, 3, 5, 1)).reshape(B, 128, 128, 96)
    pf = jnp.concatenate([nhwc(pha), nhwc(fgr)], axis=-1)
    pfp = jnp.pad(pf, ((0, 0), (1, 1), (1, 1), (0, 0)), mode='edge')
    pft = jnp.stack([pfp[:, 16 * t:16 * t + 18] for t in range(8)], axis=1)

    w1, b1 = _fold_bn(conv1_w, bn1_g, bn1_b, bn1_m, bn1_v)
    w2, b2 = _fold_bn(conv2_w, bn2_g, bn2_b, bn2_m, bn2_v)
    w3, b3 = _fold_bn(conv3_w, bn3_g, bn3_b, bn3_m, bn3_v)
    W1 = _poly_w(w1, _A1, 2)
    W2 = _poly_w(w2, _A2, 2)
    W3f = _poly_w(w3, _A3, 4)
    W3 = jnp.einsum('rs,tsd->trd', _F3, W3f)
    W4 = _poly_w(conv4_w, _A4, 4)
    bb1 = jnp.tile(b1, 4)[None]
    bb2 = jnp.tile(b2, 4)[None]
    bb3 = jnp.tile(b3, 16)[None]
    bb4 = jnp.tile(conv4_b, 16)[None]

    refm = pl.pallas_call(
        _topk_kernel,
        grid=(B,),
        in_specs=[pl.BlockSpec((1, 128, 128), lambda b: (b, 0, 0))],
        out_specs=pl.BlockSpec((1, 128, 128), lambda b: (b, 0, 0)),
        out_shape=jax.ShapeDtypeStruct((B, 128, 128), f32),
    )(err.reshape(B, 128, 128))

    xyc = pl.pallas_call(
        _prep_kernel,
        grid=(B, 8),
        in_specs=[
            pl.BlockSpec((1, 1, 18, 130, 36), lambda b, t: (b, t, 0, 0, 0)),
            pl.BlockSpec((1, 16, 128, 96), lambda b, t: (b, t, 0, 0)),
        ],
        out_specs=pl.BlockSpec((1, 16, 128, 168), lambda b, t: (b, t, 0, 0)),
        out_shape=jax.ShapeDtypeStruct((B, 128, 128, 168), f32),
    )(x36t, sb16)

    fpad = lambda a: jnp.pad(a, ((0, 0), (0, FL - NT * TILE), (0, 0)))
    h1 = _conv_call(_flat(xyc, 2, 2), W1, bb1, 96, True)
    h2 = _conv_call(fpad(h1), W2, bb2, 64, True)
    sbf = _flat(sb16, 1, 1)
    h2f = fpad(h2)
    h3 = pl.pallas_call(
        _mk_conv(True, True),
        grid=(B, NT),
        in_specs=[
            pl.BlockSpec((1, 1, TIN, 64), lambda b, t: (b, t, 0, 0)),
            pl.BlockSpec((1, 1, TIN, 96), lambda b, t: (b, t, 0, 0)),
            pl.BlockSpec((4, 64, 192), lambda b, t: (0, 0, 0)),
            pl.BlockSpec((4, 96, 192), lambda b, t: (0, 0, 0)),
            pl.BlockSpec((1, 192), lambda b, t: (0, 0)),
        ],
        out_specs=pl.BlockSpec((1, TILE, 192), lambda b, t: (b, t, 0)),
        out_shape=jax.ShapeDtypeStruct((B, NT * TILE, 192), f32),
    )(_tiles(h2f), _tiles(sbf), W3[:, :64, :], W3[:, 64:, :], bb3)
    h4 = _conv_call(fpad(h3), W4, bb4, 64, False)

    h4g = h4[:, :133 * 132, :].reshape(B, 133, 132, 64)
    out64 = pl.pallas_call(
        _merge_kernel,
        grid=(B, 8),
        in_specs=[
            pl.BlockSpec((1, 16, 132, 64), lambda b, t: (b, t, 0, 0)),
            pl.BlockSpec((1, 1, 18, 130, 4), lambda b, t: (b, t, 0, 0, 0)),
            pl.BlockSpec((1, 16, 128, 1), lambda b, t: (b, t, 0, 0)),
        ],
        out_specs=pl.BlockSpec((1, 16, 128, 64), lambda b, t: (b, t, 0, 0)),
        out_shape=jax.ShapeDtypeStruct((B, 128, 128, 64), f32),
    )(h4g, pft, refm[:, :, :, None])

    out = jnp.transpose(out64.reshape(B, 128, 128, 4, 4, 4),
                        (0, 5, 1, 3, 2, 4)).reshape(B, 4, 512, 512)
    return (out[:, :1], out[:, 1:], refm[:, None])


# speedup vs baseline: 304.4031x; 1.0503x over previous
import numpy as np
import jax
import jax.numpy as jnp
from jax.experimental import pallas as pl

BNEPS = 1e-05
KSEL = 5000
NT = 4
TILE = 4424
FL = 17832


def _fold_bn(w, g, b, m, v):
    s = g / jnp.sqrt(v + BNEPS)
    return w * s[:, None, None, None], b - m * s


def _poly_sel(f, o):
    A = np.zeros((2, 2, f, f, f, f, 3, 3), np.float32)
    smin = o // f
    for py in range(f):
        for dy in range(3):
            vy = py + dy + o
            sy, qy = vy // f - smin, vy % f
            for px in range(f):
                for dx in range(3):
                    vx = px + dx + o
                    sx, qx = vx // f - smin, vx % f
                    A[sy, sx, qy, qx, py, px, dy, dx] = 1.0
    return A.reshape(4, f, f, f, f, 3, 3)


_A1 = _poly_sel(2, -2)
_A2 = _poly_sel(2, 0)
_A3 = _poly_sel(4, -2)
_A4 = _poly_sel(4, 0)
_F3 = np.zeros((160, 352), np.float32)
for _qy in range(4):
    for _qx in range(4):
        _r = (_qy * 4 + _qx) * 22
        for _c in range(16):
            _F3[16 * (2 * (_qy // 2) + (_qx // 2)) + _c, _r + _c] = 1.0
        for _c in range(6):
            _F3[64 + 6 * (4 * _qy + _qx) + _c, _r + 16 + _c] = 1.0
_F3 = jnp.asarray(_F3)


def _poly_w(wf, A, f):
    W = jnp.einsum('tabcdyx,oiyx->tabicdo', jnp.asarray(A), wf)
    return W.reshape(4, f * f * wf.shape[1], f * f * wf.shape[0])


def _flat(content, rpad, cpad):
    B, H, W, C = content.shape
    x = jnp.pad(content, ((0, 0), (rpad, 133 - rpad - H), (cpad, 132 - cpad - W), (0, 0)))
    x = x.reshape(B, 133 * 132, C)
    return jnp.pad(x, ((0, 0), (0, FL - 133 * 132), (0, 0)))


def _topk_kernel(e_ref, o_ref):
    e = e_ref[0]
    bits = jax.lax.bitcast_convert_type(e, jnp.int32)

    def body(i, lohi):
        lo, hi = lohi
        mid = lo + (hi - lo) // 2
        cnt = jnp.sum((bits > mid).astype(jnp.int32))
        big = cnt >= KSEL
        return jnp.where(big, mid + 1, lo), jnp.where(big, hi, mid)

    t, _ = jax.lax.fori_loop(0, 31, body, (jnp.int32(0), jnp.int32(2**31 - 1)))
    gt = bits > t
    eq = bits == t
    need = (KSEL - jnp.sum(gt.astype(jnp.int32))).astype(jnp.float32)
    eqf = eq.astype(jnp.float32)
    row = jax.lax.broadcasted_iota(jnp.int32, (128, 128), 0)
    col = jax.lax.broadcasted_iota(jnp.int32, (128, 128), 1)
    lt = (row < col).astype(jnp.float32)
    sl = (row > col).astype(jnp.float32)
    in_row = jax.lax.dot_general(eqf, lt, (((1,), (0,)), ((), ())),
                                 preferred_element_type=jnp.float32)
    rowcnt = jnp.sum(eqf, axis=1, keepdims=True)
    rowpref = jax.lax.dot_general(sl, rowcnt, (((1,), (0,)), ((), ())),
                                  preferred_element_type=jnp.float32)
    prefix = in_row + rowpref
    mask = gt | (eq & (prefix < need))
    o_ref[0] = jnp.where(mask & (bits > 0), 1.0, 0.0)


def _prep_kernel(xp_ref, sb_ref, o_ref):
    Pw = xp_ref[0, 0]
    r0 = 0.25 * Pw[0:16] + 0.75 * Pw[1:17]
    r1 = 0.75 * Pw[1:17] + 0.25 * Pw[2:18]
    S = sb_ref[0]
    for qy in range(2):
        r = (r0, r1)[qy]
        c0 = 0.25 * r[:, 0:128, :] + 0.75 * r[:, 1:129, :]
        c1 = 0.75 * r[:, 1:129, :] + 0.25 * r[:, 2:130, :]
        for qx in range(2):
            u = (c0, c1)[qx]
            acc = None
            for a in range(2):
                for b in range(2):
                    cc = ((2 * qy + a) * 4 + (2 * qx + b)) * 6
                    piece = S[:, :, cc:cc + 6]
                    acc = piece if acc is None else acc + piece
            base = (qy * 2 + qx) * 42
            o_ref[0, :, :, base:base + 36] = u
            o_ref[0, :, :, base + 36:base + 42] = 0.25 * acc


def _dot(x, w):
    return jax.lax.dot_general(x, w, (((1,), (0,)), ((), ())),
                               preferred_element_type=jnp.float32)


def _mk_conv(relu, two_in):
    if two_in:
        def k(xa_ref, xb_ref, wa_ref, wb_ref, b_ref, o_ref):
            for t in range(NT):
                s = t * TILE
                acc = None
                for j, off in enumerate((0, 1, 132, 133)):
                    p = _dot(xa_ref[0, s + off:s + off + TILE, :], wa_ref[j])
                    p = p + _dot(xb_ref[0, s + off:s + off + TILE, :], wb_ref[j])
                    acc = p if acc is None else acc + p
                acc = acc + b_ref[0]
                o_ref[0, s:s + TILE, :] = jnp.maximum(acc, 0.0) if relu else acc
        return k

    def k(x_ref, w_ref, b_ref, o_ref):
        for t in range(NT):
            s = t * TILE
            acc = None
            for j, off in enumerate((0, 1, 132, 133)):
                p = _dot(x_ref[0, s + off:s + off + TILE, :], w_ref[j])
                acc = p if acc is None else acc + p
            acc = acc + b_ref[0]
            o_ref[0, s:s + TILE, :] = jnp.maximum(acc, 0.0) if relu else acc
    return k


def _merge_kernel(h4_ref, pf_ref, ref_ref, o_ref):
    h4 = h4_ref[0, :, 0:128, :]
    P = pf_ref[0, 0]
    b0 = 0.375 * P[0:16] + 0.625 * P[1:17]
    b1 = 0.125 * P[0:16] + 0.875 * P[1:17]
    b2 = 0.875 * P[1:17] + 0.125 * P[2:18]
    b3 = 0.625 * P[1:17] + 0.375 * P[2:18]
    sel = ref_ref[0] > 0.0
    for ty in range(4):
        r = (b0, b1, b2, b3)[ty]
        cs = (0.375 * r[:, 0:128, :] + 0.625 * r[:, 1:129, :],
              0.125 * r[:, 0:128, :] + 0.875 * r[:, 1:129, :],
              0.875 * r[:, 1:129, :] + 0.125 * r[:, 2:130, :],
              0.625 * r[:, 1:129, :] + 0.375 * r[:, 2:130, :])
        for tx in range(4):
            cc = (ty * 4 + tx) * 4
            o_ref[0, :, :, cc:cc + 4] = jnp.where(sel, h4[:, :, cc:cc + 4], cs[tx])


def _conv_call(xf, W, bias, cout, relu):
    B = xf.shape[0]
    return pl.pallas_call(
        _mk_conv(relu, False),
        grid=(B,),
        in_specs=[
            pl.BlockSpec((1, FL, xf.shape[2]), lambda b: (b, 0, 0)),
            pl.BlockSpec(W.shape, lambda b: (0, 0, 0)),
            pl.BlockSpec((1, cout), lambda b: (0, 0)),
        ],
        out_specs=pl.BlockSpec((1, FL, cout), lambda b: (b, 0, 0)),
        out_shape=jax.ShapeDtypeStruct((B, FL, cout), jnp.float32),
    )(xf, W, bias)


def _conv2_call(xa, xb, Wa, Wb, bias, cout, relu):
    B = xa.shape[0]
    return pl.pallas_call(
        _mk_conv(relu, True),
        grid=(B,),
        in_specs=[
            pl.BlockSpec((1, FL, xa.shape[2]), lambda b: (b, 0, 0)),
            pl.BlockSpec((1, FL, xb.shape[2]), lambda b: (b, 0, 0)),
            pl.BlockSpec(Wa.shape, lambda b: (0, 0, 0)),
            pl.BlockSpec(Wb.shape, lambda b: (0, 0, 0)),
            pl.BlockSpec((1, cout), lambda b: (0, 0)),
        ],
        out_specs=pl.BlockSpec((1, FL, cout), lambda b: (b, 0, 0)),
        out_shape=jax.ShapeDtypeStruct((B, FL, cout), jnp.float32),
    )(xa, xb, Wa, Wb, bias)


def kernel(src, bgr, pha, fgr, err, hid, conv1_w, bn1_g, bn1_b, bn1_m, bn1_v,
           conv2_w, bn2_g, bn2_b, bn2_m, bn2_v, conv3_w, bn3_g, bn3_b, bn3_m, bn3_v,
           conv4_w, conv4_b):
    B = src.shape[0]
    f32 = jnp.float32
    nhwc = lambda a: jnp.transpose(a, (0, 2, 3, 1))

    x36 = jnp.concatenate([nhwc(hid), nhwc(pha), nhwc(fgr)], axis=-1)
    x36p = jnp.pad(x36, ((0, 0), (1, 1), (1, 1), (0, 0)), mode='edge')
    x36t = jnp.stack([x36p[:, 16 * t:16 * t + 18] for t in range(8)], axis=1)
    sb = jnp.concatenate([src, bgr], axis=1)
    sb16 = jnp.transpose(sb.reshape(B, 6, 128, 4, 128, 4),
                         (0, 2, 4, 3, 5, 1)).reshape(B, 128, 128, 96)
    pf = jnp.concatenate([nhwc(pha), nhwc(fgr)], axis=-1)
    pfp = jnp.pad(pf, ((0, 0), (1, 1), (1, 1), (0, 0)), mode='edge')
    pft = jnp.stack([pfp[:, 16 * t:16 * t + 18] for t in range(8)], axis=1)

    w1, b1 = _fold_bn(conv1_w, bn1_g, bn1_b, bn1_m, bn1_v)
    w2, b2 = _fold_bn(conv2_w, bn2_g, bn2_b, bn2_m, bn2_v)
    w3, b3 = _fold_bn(conv3_w, bn3_g, bn3_b, bn3_m, bn3_v)
    W1 = _poly_w(w1, _A1, 2)
    W2 = _poly_w(w2, _A2, 2)
    W3f = _poly_w(w3, _A3, 4)
    W3 = jnp.einsum('rs,tsd->trd', _F3, W3f)
    W4 = _poly_w(conv4_w, _A4, 4)
    bb1 = jnp.tile(b1, 4)[None]
    bb2 = jnp.tile(b2, 4)[None]
    bb3 = jnp.tile(b3, 16)[None]
    bb4 = jnp.tile(conv4_b, 16)[None]

    refm = pl.pallas_call(
        _topk_kernel,
        grid=(B,),
        in_specs=[pl.BlockSpec((1, 128, 128), lambda b: (b, 0, 0))],
        out_specs=pl.BlockSpec((1, 128, 128), lambda b: (b, 0, 0)),
        out_shape=jax.ShapeDtypeStruct((B, 128, 128), f32),
    )(err.reshape(B, 128, 128))

    xyc = pl.pallas_call(
        _prep_kernel,
        grid=(B, 8),
        in_specs=[
            pl.BlockSpec((1, 1, 18, 130, 36), lambda b, t: (b, t, 0, 0, 0)),
            pl.BlockSpec((1, 16, 128, 96), lambda b, t: (b, t, 0, 0)),
        ],
        out_specs=pl.BlockSpec((1, 16, 128, 168), lambda b, t: (b, t, 0, 0)),
        out_shape=jax.ShapeDtypeStruct((B, 128, 128, 168), f32),
    )(x36t, sb16)

    h1 = _conv_call(_flat(xyc, 2, 2), W1, bb1, 96, True)
    h2 = _conv_call(h1, W2, bb2, 64, True)
    sbf = _flat(sb16, 1, 1)
    h3a = _conv2_call(h2, sbf, W3[:, :64, :96], W3[:, 64:, :96], bb3[:, :96], 96, True)
    h3b = _conv2_call(h2, sbf, W3[:, :64, 96:], W3[:, 64:, 96:], bb3[:, 96:], 96, True)
    h4 = _conv2_call(h3a, h3b, W4[:, :96, :], W4[:, 96:, :], bb4, 64, False)

    h4g = h4[:, :133 * 132, :].reshape(B, 133, 132, 64)
    out64 = pl.pallas_call(
        _merge_kernel,
        grid=(B, 8),
        in_specs=[
            pl.BlockSpec((1, 16, 132, 64), lambda b, t: (b, t, 0, 0)),
            pl.BlockSpec((1, 1, 18, 130, 4), lambda b, t: (b, t, 0, 0, 0)),
            pl.BlockSpec((1, 16, 128, 1), lambda b, t: (b, t, 0, 0)),
        ],
        out_specs=pl.BlockSpec((1, 16, 128, 64), lambda b, t: (b, t, 0, 0)),
        out_shape=jax.ShapeDtypeStruct((B, 128, 128, 64), f32),
    )(h4g, pft, refm[:, :, :, None])

    out = jnp.transpose(out64.reshape(B, 128, 128, 4, 4, 4),
                        (0, 5, 1, 3, 2, 4)).reshape(B, 4, 512, 512)
    return (out[:, :1], out[:, 1:], refm[:, None])
